# Initial kernel scaffold; baseline (speedup 1.0000x reference)
import jax
import jax.numpy as jnp
from jax.experimental import pallas as pl


def kernel(x, ts2img_weights):
    raise NotImplementedError("write your pallas kernel here")


# speedup vs baseline: 1.9505x; 1.9505x over previous
import functools

import jax
import jax.numpy as jnp
from jax import lax
from jax.experimental import pallas as pl
from jax.experimental.pallas import tpu as pltpu
from jax.experimental.pallas import tpu_sc as plsc

B, L, D, M, S = 64, 512, 8, 6, 64
NC, NS, LANES = 2, 16, 16
NW = NC * NS
B_PER_W = B // NW
NCH = S // LANES


def _rsqrt(a):
    bits = plsc.bitcast(a, jnp.int32)
    r = plsc.bitcast(jnp.int32(0x5F3759DF) - lax.shift_right_logical(bits, 1),
                     jnp.float32)
    for _ in range(3):
        r = r * (1.5 - 0.5 * a * r * r)
    return r


def _body(x_hbm, wpad_hbm, lo_hbm, hi_hbm, rw_hbm, out_hbm,
          lo_v, hi_v, rw_v, xb_v, wb_v, raw_v, s_v, q_v, a_v, b_v, c_v, img_v):
    wid = lax.axis_index("s") * NC + lax.axis_index("c")
    pltpu.sync_copy(lo_hbm, lo_v)
    pltpu.sync_copy(hi_hbm, hi_v)
    pltpu.sync_copy(rw_hbm, rw_v)
    lane = lax.iota(jnp.int32, LANES)

    def b_loop(bi, _):
        b = wid * B_PER_W + bi
        pltpu.sync_copy(x_hbm.at[b], xb_v)
        pltpu.sync_copy(wpad_hbm.at[b], wb_v)

        def d_loop(d, _):
            dsplat = jnp.full((LANES,), 0, jnp.int32) + d
            mnv = jnp.full((LANES,), jnp.inf, jnp.float32)
            mxv = jnp.full((LANES,), -jnp.inf, jnp.float32)
            for c in range(NCH):
                sl = pl.ds(c * LANES, LANES)
                slo = plsc.load_gather(xb_v, [lo_v[sl] * D + dsplat])
                shi = plsc.load_gather(xb_v, [hi_v[sl] * D + dsplat])
                wr = rw_v[sl]
                raw = slo * (1.0 - wr) + shi * wr
                raw_v[sl] = raw
                mnv = jnp.minimum(mnv, raw)
                mxv = jnp.maximum(mxv, raw)
            mn = jnp.min(mnv)
            inv = jnp.full((LANES,), 2.0, jnp.float32) / (jnp.max(mxv) - mn + 1e-8)
            for c in range(NCH):
                sl = pl.ds(c * LANES, LANES)
                s = (raw_v[sl] - mn) * inv - 1.0
                s_v[sl] = s
                scl = jnp.clip(s, -1.0 + 1e-6, 1.0 - 1e-6)
                aa = 1.0 - scl * scl
                q_v[sl] = aa * _rsqrt(aa)
            widx = jnp.minimum(d * M + lane, 63)
            wv = plsc.load_gather(wb_v, [widx])
            wv = jnp.where(lane < M, wv, -jnp.inf)
            for k in range(3):
                mx = jnp.max(wv)
                mk = jnp.min(jnp.where(wv == mx, lane, 1000))
                wv = jnp.where(lane == mk, -jnp.inf, wv)
                e0 = jnp.where(mk == 0, 1.0, 0.0)
                e1 = jnp.where(mk == 1, 1.0, 0.0)
                e2 = jnp.where(mk == 2, 1.0, 0.0)
                e4 = jnp.where(mk == 4, 1.0, 0.0)
                e5 = jnp.where(mk == 5, 1.0, 0.0)
                c0 = jnp.where(mk == 3, 1.0, 0.0) + e0 - 2.0 * e4
                c4 = -e2
                c6 = 0.5 * e5
                for c in range(NCH):
                    sl = pl.ds(c * LANES, LANES)
                    s = s_v[sl]
                    q = q_v[sl]
                    a_v[sl] = c0 * s + e1 * q
                    b_v[sl] = (-e0) * q + (-e1) * s
                    c_v[sl] = (e4 * s) * s + c6 * s

                def row(i, _):
                    ai = a_v[pl.ds(i, LANES)][0]
                    bi_ = b_v[pl.ds(i, LANES)][0]
                    ci = c_v[pl.ds(i, LANES)][0]
                    si = s_v[pl.ds(i, LANES)][0]
                    for c in range(NCH):
                        sl = pl.ds(c * LANES, LANES)
                        s = s_v[sl]
                        r = (ai * s + bi_ * q_v[sl]
                             + c4 * jnp.abs(s - si) + (c_v[sl] + ci))
                        img_v[i, sl] = r
                    return 0

                lax.fori_loop(0, S, row, 0)
                pltpu.sync_copy(img_v, out_hbm.at[b, d, k])
            return 0

        lax.fori_loop(0, D, d_loop, 0)
        return 0

    lax.fori_loop(0, B_PER_W, b_loop, 0)


@jax.jit
def kernel(x, ts2img_weights):
    f32, i32 = jnp.float32, jnp.int32
    pos = jnp.linspace(0.0, L - 1.0, S)
    lo = jnp.floor(pos).astype(i32)
    hi = jnp.clip(lo + 1, 0, L - 1)
    rw = (pos - lo.astype(pos.dtype)).astype(f32)
    wpad = jnp.zeros((B, 64), f32).at[:, : D * M].set(
        ts2img_weights.reshape(B, D * M))

    run = pl.kernel(
        _body,
        out_type=jax.ShapeDtypeStruct((B, D, 3, S, S), f32),
        mesh=plsc.VectorSubcoreMesh(
            core_axis_name="c", subcore_axis_name="s",
            num_cores=NC, num_subcores=NS),
        compiler_params=pltpu.CompilerParams(needs_layout_passes=False),
        scratch_types=[
            pltpu.VMEM((S,), i32),
            pltpu.VMEM((S,), i32),
            pltpu.VMEM((S,), f32),
            pltpu.VMEM((L * D,), f32),
            pltpu.VMEM((64,), f32),
            pltpu.VMEM((S,), f32),
            pltpu.VMEM((S + LANES,), f32),
            pltpu.VMEM((S,), f32),
            pltpu.VMEM((S + LANES,), f32),
            pltpu.VMEM((S + LANES,), f32),
            pltpu.VMEM((S + LANES,), f32),
            pltpu.VMEM((S, S), f32),
        ],
    )
    return run(x.reshape(B, L * D), wpad, lo, hi, rw)


# speedup vs baseline: 3.8664x; 1.9822x over previous
import jax
import jax.numpy as jnp
from jax import lax
from jax.experimental import pallas as pl
from jax.experimental.pallas import tpu as pltpu
from jax.experimental.pallas import tpu_sc as plsc

B, L, D, M, S = 64, 512, 8, 6, 64
NC, NS, LANES = 2, 16, 16
NW = NC * NS
B_PER_W = B // NW
NCH = S // LANES
NRB = S // LANES


def _rsqrt(a):
    bits = plsc.bitcast(a, jnp.int32)
    r = plsc.bitcast(jnp.int32(0x5F3759DF) - lax.shift_right_logical(bits, 1),
                     jnp.float32)
    for _ in range(3):
        r = r * (1.5 - 0.5 * a * r * r)
    return r


def _splat(vec, r):
    idx = jnp.full((LANES, 1), r, jnp.int32)
    return jnp.take_along_axis(vec, idx.reshape(LANES), axis=0,
                               mode="promise_in_bounds")


def _body(x_hbm, wpad_hbm, lo_hbm, hi_hbm, rw_hbm, out_hbm,
          lo_v, hi_v, rw_v, xb_v, wb_v, s_v, q_v, h_v,
          img0, img1, img2, sem0, sem1, sem2):
    wid = lax.axis_index("s") * NC + lax.axis_index("c")
    pltpu.sync_copy(lo_hbm, lo_v)
    pltpu.sync_copy(hi_hbm, hi_v)
    pltpu.sync_copy(rw_hbm, rw_v)
    lane = lax.iota(jnp.int32, LANES)
    imgs = (img0, img1, img2)
    sems = (sem0, sem1, sem2)

    def b_loop(bi, _):
        b = wid * B_PER_W + bi
        pltpu.sync_copy(x_hbm.at[b], xb_v)
        pltpu.sync_copy(wpad_hbm.at[b], wb_v)

        def d_loop(d, _):
            dsplat = jnp.full((LANES,), 0, jnp.int32) + d
            raw = []
            for c in range(NCH):
                sl = pl.ds(c * LANES, LANES)
                slo = plsc.load_gather(xb_v, [lo_v[sl] * D + dsplat])
                shi = plsc.load_gather(xb_v, [hi_v[sl] * D + dsplat])
                wr = rw_v[sl]
                raw.append(slo * (1.0 - wr) + shi * wr)
            mnv = jnp.minimum(jnp.minimum(raw[0], raw[1]),
                              jnp.minimum(raw[2], raw[3]))
            mxv = jnp.maximum(jnp.maximum(raw[0], raw[1]),
                              jnp.maximum(raw[2], raw[3]))
            mn = jnp.min(mnv)
            inv = jnp.full((LANES,), 2.0, jnp.float32) / (jnp.max(mxv) - mn + 1e-8)
            sj, qj, hj = [], [], []
            for c in range(NCH):
                sl = pl.ds(c * LANES, LANES)
                s = (raw[c] - mn) * inv - 1.0
                scl = jnp.clip(s, -1.0 + 1e-6, 1.0 - 1e-6)
                aa = 1.0 - scl * scl
                q = aa * _rsqrt(aa)
                h = 0.5 * s
                s_v[sl] = s
                q_v[sl] = q
                h_v[sl] = h
                sj.append(s)
                qj.append(q)
                hj.append(h)
            wv = plsc.load_gather(wb_v, [jnp.minimum(d * M + lane, 63)])
            wv = jnp.where(lane < M, wv, -jnp.inf)
            handles = []
            for k in range(3):
                mx = jnp.max(wv)
                mk = jnp.min(jnp.where(wv == mx, lane, 1000))
                wv = jnp.where(lane == mk, -jnp.inf, wv)
                img = imgs[k]

                def mk_branch(method, img=img):
                    def blk(rb, _):
                        base = rb * LANES
                        sl = pl.ds(base, LANES)
                        sb = s_v[sl]
                        qb = q_v[sl]
                        hb = h_v[sl]
                        for r in range(LANES):
                            i = base + r
                            if method == 0:
                                si = _splat(sb, r)
                                qi = _splat(qb, r)
                                for c in range(NCH):
                                    img[i, pl.ds(c * LANES, LANES)] = (
                                        si * sj[c] - qi * qj[c])
                            elif method == 1:
                                si = _splat(sb, r)
                                qi = _splat(qb, r)
                                for c in range(NCH):
                                    img[i, pl.ds(c * LANES, LANES)] = (
                                        qi * sj[c] - si * qj[c])
                            elif method == 2:
                                si = _splat(sb, r)
                                for c in range(NCH):
                                    img[i, pl.ds(c * LANES, LANES)] = (
                                        0.0 - jnp.abs(sj[c] - si))
                            elif method == 3:
                                si = _splat(sb, r)
                                for c in range(NCH):
                                    img[i, pl.ds(c * LANES, LANES)] = si * sj[c]
                            elif method == 4:
                                si = _splat(sb, r)
                                for c in range(NCH):
                                    t = sj[c] - si
                                    img[i, pl.ds(c * LANES, LANES)] = t * t
                            else:
                                hi_ = _splat(hb, r)
                                for c in range(NCH):
                                    img[i, pl.ds(c * LANES, LANES)] = hi_ + hj[c]
                        return 0

                    def branch():
                        lax.fori_loop(0, NRB, blk, 0)

                    return branch

                lax.switch(mk, [mk_branch(m) for m in range(M)])
                handles.append(
                    pltpu.async_copy(img, out_hbm.at[b, d, k], sems[k]))
            for hnd in handles:
                hnd.wait()
            return 0

        lax.fori_loop(0, D, d_loop, 0)
        return 0

    lax.fori_loop(0, B_PER_W, b_loop, 0)


@jax.jit
def kernel(x, ts2img_weights):
    f32, i32 = jnp.float32, jnp.int32
    pos = jnp.linspace(0.0, L - 1.0, S)
    lo = jnp.floor(pos).astype(i32)
    hi = jnp.clip(lo + 1, 0, L - 1)
    rw = (pos - lo.astype(pos.dtype)).astype(f32)
    wpad = jnp.zeros((B, 64), f32).at[:, : D * M].set(
        ts2img_weights.reshape(B, D * M))

    run = pl.kernel(
        _body,
        out_type=jax.ShapeDtypeStruct((B, D, 3, S, S), f32),
        mesh=plsc.VectorSubcoreMesh(
            core_axis_name="c", subcore_axis_name="s",
            num_cores=NC, num_subcores=NS),
        compiler_params=pltpu.CompilerParams(needs_layout_passes=False),
        scratch_types=[
            pltpu.VMEM((S,), i32),
            pltpu.VMEM((S,), i32),
            pltpu.VMEM((S,), f32),
            pltpu.VMEM((L * D,), f32),
            pltpu.VMEM((64,), f32),
            pltpu.VMEM((S,), f32),
            pltpu.VMEM((S,), f32),
            pltpu.VMEM((S,), f32),
            pltpu.VMEM((S, S), f32),
            pltpu.VMEM((S, S), f32),
            pltpu.VMEM((S, S), f32),
            pltpu.SemaphoreType.DMA,
            pltpu.SemaphoreType.DMA,
            pltpu.SemaphoreType.DMA,
        ],
    )
    return run(x.reshape(B, L * D), wpad, lo, hi, rw)


# speedup vs baseline: 4.0029x; 1.0353x over previous
import jax
import jax.numpy as jnp
from jax import lax
from jax.experimental import pallas as pl
from jax.experimental.pallas import tpu as pltpu
from jax.experimental.pallas import tpu_sc as plsc

B, L, D, M, S = 64, 512, 8, 6, 64
NC, NS, LANES = 2, 16, 16
NW = NC * NS
B_PER_W = B // NW
NCH = S // LANES
NRB = S // LANES


def _rsqrt(a):
    bits = plsc.bitcast(a, jnp.int32)
    r = plsc.bitcast(jnp.int32(0x5F3759DF) - lax.shift_right_logical(bits, 1),
                     jnp.float32)
    for _ in range(3):
        r = r * (1.5 - 0.5 * a * r * r)
    return r


def _splat(vec, r):
    idx = jnp.full((LANES, 1), r, jnp.int32)
    return jnp.take_along_axis(vec, idx.reshape(LANES), axis=0,
                               mode="promise_in_bounds")


def _body(x_hbm, wpad_hbm, lo_hbm, hi_hbm, rw_hbm, out_hbm,
          lo_v, hi_v, rw_v, xb_v, wb_v, s_v, q_v, h_v,
          img0, img1, img2, sem0, sem1, sem2):
    wid = lax.axis_index("s") * NC + lax.axis_index("c")
    pltpu.sync_copy(lo_hbm, lo_v)
    pltpu.sync_copy(hi_hbm, hi_v)
    pltpu.sync_copy(rw_hbm, rw_v)
    lane = lax.iota(jnp.int32, LANES)
    imgs = (img0, img1, img2)
    sems = (sem0, sem1, sem2)

    def b_loop(bi, _):
        b = wid * B_PER_W + bi
        pltpu.sync_copy(x_hbm.at[b], xb_v)
        pltpu.sync_copy(wpad_hbm.at[b], wb_v)

        def d_loop(d, _):
            dsplat = jnp.full((LANES,), 0, jnp.int32) + d
            raw = []
            for c in range(NCH):
                sl = pl.ds(c * LANES, LANES)
                slo = plsc.load_gather(xb_v, [lo_v[sl] * D + dsplat])
                shi = plsc.load_gather(xb_v, [hi_v[sl] * D + dsplat])
                wr = rw_v[sl]
                raw.append(slo * (1.0 - wr) + shi * wr)
            mnv = jnp.minimum(jnp.minimum(raw[0], raw[1]),
                              jnp.minimum(raw[2], raw[3]))
            mxv = jnp.maximum(jnp.maximum(raw[0], raw[1]),
                              jnp.maximum(raw[2], raw[3]))
            mn = jnp.min(mnv)
            inv = jnp.full((LANES,), 2.0, jnp.float32) / (jnp.max(mxv) - mn + 1e-8)
            sj, qj, hj = [], [], []
            for c in range(NCH):
                sl = pl.ds(c * LANES, LANES)
                s = (raw[c] - mn) * inv - 1.0
                scl = jnp.clip(s, -1.0 + 1e-6, 1.0 - 1e-6)
                aa = 1.0 - scl * scl
                q = aa * _rsqrt(aa)
                h = 0.5 * s
                s_v[sl] = s
                q_v[sl] = q
                h_v[sl] = h
                sj.append(s)
                qj.append(q)
                hj.append(h)
            wv = plsc.load_gather(wb_v, [jnp.minimum(d * M + lane, 63)])
            wv = jnp.where(lane < M, wv, -jnp.inf)
            for k in range(3):
                mx = jnp.max(wv)
                mk = jnp.min(jnp.where(wv == mx, lane, 1000))
                wv = jnp.where(lane == mk, -jnp.inf, wv)
                img = imgs[k]

                @pl.when(jnp.logical_or(bi > 0, d > 0))
                def _(img=img, k=k):
                    pltpu.make_async_copy(
                        img, out_hbm.at[b, d, k], sems[k]).wait()

                def mk_branch(method, img=img):
                    def blk(rb, _):
                        base = rb * LANES
                        sl = pl.ds(base, LANES)
                        sb = s_v[sl]
                        qb = q_v[sl]
                        hb = h_v[sl]
                        for r in range(LANES):
                            i = base + r
                            if method == 0:
                                si = _splat(sb, r)
                                qi = _splat(qb, r)
                                for c in range(NCH):
                                    img[i, pl.ds(c * LANES, LANES)] = (
                                        si * sj[c] - qi * qj[c])
                            elif method == 1:
                                si = _splat(sb, r)
                                qi = _splat(qb, r)
                                for c in range(NCH):
                                    img[i, pl.ds(c * LANES, LANES)] = (
                                        qi * sj[c] - si * qj[c])
                            elif method == 2:
                                si = _splat(sb, r)
                                for c in range(NCH):
                                    img[i, pl.ds(c * LANES, LANES)] = (
                                        0.0 - jnp.abs(sj[c] - si))
                            elif method == 3:
                                si = _splat(sb, r)
                                for c in range(NCH):
                                    img[i, pl.ds(c * LANES, LANES)] = si * sj[c]
                            elif method == 4:
                                si = _splat(sb, r)
                                for c in range(NCH):
                                    t = sj[c] - si
                                    img[i, pl.ds(c * LANES, LANES)] = t * t
                            else:
                                hi_ = _splat(hb, r)
                                for c in range(NCH):
                                    img[i, pl.ds(c * LANES, LANES)] = hi_ + hj[c]
                        return 0

                    def branch():
                        lax.fori_loop(0, NRB, blk, 0)

                    return branch

                lax.switch(mk, [mk_branch(m) for m in range(M)])
                pltpu.async_copy(img, out_hbm.at[b, d, k], sems[k])
            return 0

        lax.fori_loop(0, D, d_loop, 0)
        return 0

    lax.fori_loop(0, B_PER_W, b_loop, 0)
    last = NW * B_PER_W - 1
    for k in range(3):
        pltpu.make_async_copy(imgs[k], out_hbm.at[last, D - 1, k],
                              sems[k]).wait()


@jax.jit
def kernel(x, ts2img_weights):
    f32, i32 = jnp.float32, jnp.int32
    pos = jnp.linspace(0.0, L - 1.0, S)
    lo = jnp.floor(pos).astype(i32)
    hi = jnp.clip(lo + 1, 0, L - 1)
    rw = (pos - lo.astype(pos.dtype)).astype(f32)
    wpad = jnp.zeros((B, 64), f32).at[:, : D * M].set(
        ts2img_weights.reshape(B, D * M))

    run = pl.kernel(
        _body,
        out_type=jax.ShapeDtypeStruct((B, D, 3, S, S), f32),
        mesh=plsc.VectorSubcoreMesh(
            core_axis_name="c", subcore_axis_name="s",
            num_cores=NC, num_subcores=NS),
        compiler_params=pltpu.CompilerParams(needs_layout_passes=False),
        scratch_types=[
            pltpu.VMEM((S,), i32),
            pltpu.VMEM((S,), i32),
            pltpu.VMEM((S,), f32),
            pltpu.VMEM((L * D,), f32),
            pltpu.VMEM((64,), f32),
            pltpu.VMEM((S,), f32),
            pltpu.VMEM((S,), f32),
            pltpu.VMEM((S,), f32),
            pltpu.VMEM((S, S), f32),
            pltpu.VMEM((S, S), f32),
            pltpu.VMEM((S, S), f32),
            pltpu.SemaphoreType.DMA,
            pltpu.SemaphoreType.DMA,
            pltpu.SemaphoreType.DMA,
        ],
    )
    return run(x.reshape(B, L * D), wpad, lo, hi, rw)


# speedup vs baseline: 4.0552x; 1.0131x over previous
import jax
import jax.numpy as jnp
from jax import lax
from jax.experimental import pallas as pl
from jax.experimental.pallas import tpu as pltpu
from jax.experimental.pallas import tpu_sc as plsc

B, L, D, M, S = 64, 512, 8, 6, 64
NC, NS, LANES = 2, 16, 16
NW = NC * NS
B_PER_W = B // NW
NCH = S // LANES
NRB = S // LANES


def _rsqrt(a):
    bits = plsc.bitcast(a, jnp.int32)
    r = plsc.bitcast(jnp.int32(0x5F3759DF) - lax.shift_right_logical(bits, 1),
                     jnp.float32)
    for _ in range(3):
        r = r * (1.5 - 0.5 * a * r * r)
    return r


def _splat(vec, r):
    idx = jnp.full((LANES, 1), r, jnp.int32)
    return jnp.take_along_axis(vec, idx.reshape(LANES), axis=0,
                               mode="promise_in_bounds")


def _body(x_hbm, wpad_hbm, lo_hbm, hi_hbm, rw_hbm, out_hbm,
          lo_v, hi_v, rw_v, xb_v, wb_v, s_v, q_v, h_v, imgbuf, sem):
    wid = lax.axis_index("s") * NC + lax.axis_index("c")
    pltpu.sync_copy(lo_hbm, lo_v)
    pltpu.sync_copy(hi_hbm, hi_v)
    pltpu.sync_copy(rw_hbm, rw_v)
    lane = lax.iota(jnp.int32, LANES)

    def b_loop(bi, _):
        b = wid * B_PER_W + bi
        pltpu.sync_copy(x_hbm.at[b], xb_v)
        pltpu.sync_copy(wpad_hbm.at[b], wb_v)

        def d_loop(d, _):
            dsplat = jnp.full((LANES,), 0, jnp.int32) + d
            raw = []
            for c in range(NCH):
                sl = pl.ds(c * LANES, LANES)
                slo = plsc.load_gather(xb_v, [lo_v[sl] * D + dsplat])
                shi = plsc.load_gather(xb_v, [hi_v[sl] * D + dsplat])
                wr = rw_v[sl]
                raw.append(slo * (1.0 - wr) + shi * wr)
            mnv = jnp.minimum(jnp.minimum(raw[0], raw[1]),
                              jnp.minimum(raw[2], raw[3]))
            mxv = jnp.maximum(jnp.maximum(raw[0], raw[1]),
                              jnp.maximum(raw[2], raw[3]))
            mn = jnp.min(mnv)
            inv = jnp.full((LANES,), 2.0, jnp.float32) / (jnp.max(mxv) - mn + 1e-8)
            sj, qj, hj = [], [], []
            for c in range(NCH):
                sl = pl.ds(c * LANES, LANES)
                s = (raw[c] - mn) * inv - 1.0
                scl = jnp.clip(s, -1.0 + 1e-6, 1.0 - 1e-6)
                aa = 1.0 - scl * scl
                q = aa * _rsqrt(aa)
                h = 0.5 * s
                s_v[sl] = s
                q_v[sl] = q
                h_v[sl] = h
                sj.append(s)
                qj.append(q)
                hj.append(h)
            wv = plsc.load_gather(wb_v, [jnp.minimum(d * M + lane, 63)])
            wv = jnp.where(lane < M, wv, -jnp.inf)
            t = bi * D + d
            p = t % 2
            @pl.when(t >= 2)
            def _():
                pltpu.make_async_copy(
                    imgbuf.at[p], out_hbm.at[b, d], sem).wait()

            for k in range(3):
                mx = jnp.max(wv)
                mk = jnp.min(jnp.where(wv == mx, lane, 1000))
                wv = jnp.where(lane == mk, -jnp.inf, wv)

                def mk_branch(method, k=k):
                    def blk(rb, _):
                        base = rb * LANES
                        sl = pl.ds(base, LANES)
                        sb = s_v[sl]
                        qb = q_v[sl]
                        hb = h_v[sl]
                        for r in range(LANES):
                            i = base + r
                            if method == 0:
                                si = _splat(sb, r)
                                qi = _splat(qb, r)
                                for c in range(NCH):
                                    imgbuf[p, k, i, pl.ds(c * LANES, LANES)] = (
                                        si * sj[c] - qi * qj[c])
                            elif method == 1:
                                si = _splat(sb, r)
                                qi = _splat(qb, r)
                                for c in range(NCH):
                                    imgbuf[p, k, i, pl.ds(c * LANES, LANES)] = (
                                        qi * sj[c] - si * qj[c])
                            elif method == 2:
                                si = _splat(sb, r)
                                for c in range(NCH):
                                    imgbuf[p, k, i, pl.ds(c * LANES, LANES)] = (
                                        0.0 - jnp.abs(sj[c] - si))
                            elif method == 3:
                                si = _splat(sb, r)
                                for c in range(NCH):
                                    imgbuf[p, k, i, pl.ds(c * LANES, LANES)] = (
                                        si * sj[c])
                            elif method == 4:
                                si = _splat(sb, r)
                                for c in range(NCH):
                                    tt = sj[c] - si
                                    imgbuf[p, k, i, pl.ds(c * LANES, LANES)] = (
                                        tt * tt)
                            else:
                                hi_ = _splat(hb, r)
                                for c in range(NCH):
                                    imgbuf[p, k, i, pl.ds(c * LANES, LANES)] = (
                                        hi_ + hj[c])
                        return 0

                    def branch():
                        lax.fori_loop(0, NRB, blk, 0)

                    return branch

                lax.switch(mk, [mk_branch(m) for m in range(M)])
            pltpu.async_copy(imgbuf.at[p], out_hbm.at[b, d], sem)
            return 0

        lax.fori_loop(0, D, d_loop, 0)
        return 0

    lax.fori_loop(0, B_PER_W, b_loop, 0)
    last = NW * B_PER_W - 1
    for pp in range(2):
        pltpu.make_async_copy(imgbuf.at[pp], out_hbm.at[last, D - 1],
                              sem).wait()


@jax.jit
def kernel(x, ts2img_weights):
    f32, i32 = jnp.float32, jnp.int32
    pos = jnp.linspace(0.0, L - 1.0, S)
    lo = jnp.floor(pos).astype(i32)
    hi = jnp.clip(lo + 1, 0, L - 1)
    rw = (pos - lo.astype(pos.dtype)).astype(f32)
    wpad = jnp.zeros((B, 64), f32).at[:, : D * M].set(
        ts2img_weights.reshape(B, D * M))

    run = pl.kernel(
        _body,
        out_type=jax.ShapeDtypeStruct((B, D, 3, S, S), f32),
        mesh=plsc.VectorSubcoreMesh(
            core_axis_name="c", subcore_axis_name="s",
            num_cores=NC, num_subcores=NS),
        compiler_params=pltpu.CompilerParams(needs_layout_passes=False),
        scratch_types=[
            pltpu.VMEM((S,), i32),
            pltpu.VMEM((S,), i32),
            pltpu.VMEM((S,), f32),
            pltpu.VMEM((L * D,), f32),
            pltpu.VMEM((64,), f32),
            pltpu.VMEM((S,), f32),
            pltpu.VMEM((S,), f32),
            pltpu.VMEM((S,), f32),
            pltpu.VMEM((2, 3, S, S), f32),
            pltpu.SemaphoreType.DMA,
        ],
    )
    return run(x.reshape(B, L * D), wpad, lo, hi, rw)


# speedup vs baseline: 6.8312x; 1.6846x over previous
import jax
import jax.numpy as jnp
from jax import lax
from jax.experimental import pallas as pl
from jax.experimental.pallas import tpu as pltpu
from jax.experimental.pallas import tpu_sc as plsc

B, L, D, M, S = 64, 512, 8, 6, 64
NC, NS, LANES = 2, 16, 16
NW = NC * NS
B_PER_W = B // NW
NCH = S // LANES
NRB = S // LANES
RPB = 16


def _rsqrt(a):
    bits = plsc.bitcast(a, jnp.int32)
    r = plsc.bitcast(jnp.int32(0x5F3759DF) - lax.shift_right_logical(bits, 1),
                     jnp.float32)
    for _ in range(3):
        r = r * (1.5 - 0.5 * a * r * r)
    return r


def _splat(vec, r):
    idx = jnp.full((LANES, 1), r, jnp.int32)
    return jnp.take_along_axis(vec, idx.reshape(LANES), axis=0,
                               mode="promise_in_bounds")


def _body(x_hbm, wpad_hbm, lo_hbm, hi_hbm, rw_hbm, out_hbm,
          lo_v, hi_v, rw_v, xb_v, wb_v, s_v, q_v, h_v, imgbuf, sem):
    wid = lax.axis_index("s") * NC + lax.axis_index("c")
    pltpu.sync_copy(lo_hbm, lo_v)
    pltpu.sync_copy(hi_hbm, hi_v)
    pltpu.sync_copy(rw_hbm, rw_v)
    lane = lax.iota(jnp.int32, LANES)

    def b_loop(bi, _):
        b = wid * B_PER_W + bi
        pltpu.sync_copy(x_hbm.at[b], xb_v)
        pltpu.sync_copy(wpad_hbm.at[b], wb_v)

        def d_loop(d, _):
            dsplat = jnp.full((LANES,), 0, jnp.int32) + d
            raw = []
            for c in range(NCH):
                sl = pl.ds(c * LANES, LANES)
                slo = plsc.load_gather(xb_v, [lo_v[sl] * D + dsplat])
                shi = plsc.load_gather(xb_v, [hi_v[sl] * D + dsplat])
                wr = rw_v[sl]
                raw.append(slo * (1.0 - wr) + shi * wr)
            mnv = jnp.minimum(jnp.minimum(raw[0], raw[1]),
                              jnp.minimum(raw[2], raw[3]))
            mxv = jnp.maximum(jnp.maximum(raw[0], raw[1]),
                              jnp.maximum(raw[2], raw[3]))
            mn = jnp.min(mnv)
            inv = jnp.full((LANES,), 2.0, jnp.float32) / (jnp.max(mxv) - mn + 1e-8)
            sj, qj, hj = [], [], []
            for c in range(NCH):
                sl = pl.ds(c * LANES, LANES)
                s = (raw[c] - mn) * inv - 1.0
                scl = jnp.clip(s, -1.0 + 1e-6, 1.0 - 1e-6)
                aa = 1.0 - scl * scl
                q = aa * _rsqrt(aa)
                h = 0.5 * s
                s_v[sl] = s
                q_v[sl] = q
                h_v[sl] = h
                sj.append(s)
                qj.append(q)
                hj.append(h)
            wv = plsc.load_gather(wb_v, [jnp.minimum(d * M + lane, 63)])
            wv = jnp.where(lane < M, wv, -jnp.inf)
            t = bi * D + d
            p = t % 2
            @pl.when(t >= 2)
            def _():
                pltpu.make_async_copy(
                    imgbuf.at[p], out_hbm.at[b, d], sem).wait()

            def k_loop(k, wv):
                mx = jnp.max(wv)
                mk = jnp.min(jnp.where(wv == mx, lane, 1000))
                wv = jnp.where(lane == mk, -jnp.inf, wv)

                def mk_branch(method):
                    def blk(rb, _):
                        base = rb * RPB
                        sl = pl.ds(base, LANES)
                        sb = s_v[sl]
                        qb = q_v[sl]
                        hb = h_v[sl]
                        for r in range(RPB):
                            i = base + r
                            if method == 0:
                                si = _splat(sb, r)
                                qi = _splat(qb, r)
                                for c in range(NCH):
                                    imgbuf[p, k, i, pl.ds(c * LANES, LANES)] = (
                                        si * sj[c] - qi * qj[c])
                            elif method == 1:
                                si = _splat(sb, r)
                                qi = _splat(qb, r)
                                for c in range(NCH):
                                    imgbuf[p, k, i, pl.ds(c * LANES, LANES)] = (
                                        qi * sj[c] - si * qj[c])
                            elif method == 2:
                                si = _splat(sb, r)
                                for c in range(NCH):
                                    imgbuf[p, k, i, pl.ds(c * LANES, LANES)] = (
                                        0.0 - jnp.abs(sj[c] - si))
                            elif method == 3:
                                si = _splat(sb, r)
                                for c in range(NCH):
                                    imgbuf[p, k, i, pl.ds(c * LANES, LANES)] = (
                                        si * sj[c])
                            elif method == 4:
                                si = _splat(sb, r)
                                for c in range(NCH):
                                    tt = sj[c] - si
                                    imgbuf[p, k, i, pl.ds(c * LANES, LANES)] = (
                                        tt * tt)
                            else:
                                hi_ = _splat(hb, r)
                                for c in range(NCH):
                                    imgbuf[p, k, i, pl.ds(c * LANES, LANES)] = (
                                        hi_ + hj[c])
                        return 0

                    def branch():
                        plsc.parallel_loop(0, S // RPB)(
                            lambda rb: blk(rb, 0) and None)

                    return branch

                lax.switch(mk, [mk_branch(m) for m in range(M)])
                return wv

            lax.fori_loop(0, 3, k_loop, wv)
            pltpu.async_copy(imgbuf.at[p], out_hbm.at[b, d], sem)
            return 0

        lax.fori_loop(0, D, d_loop, 0)
        return 0

    lax.fori_loop(0, B_PER_W, b_loop, 0)
    last = NW * B_PER_W - 1
    for pp in range(2):
        pltpu.make_async_copy(imgbuf.at[pp], out_hbm.at[last, D - 1],
                              sem).wait()


@jax.jit
def kernel(x, ts2img_weights):
    f32, i32 = jnp.float32, jnp.int32
    pos = jnp.linspace(0.0, L - 1.0, S)
    lo = jnp.floor(pos).astype(i32)
    hi = jnp.clip(lo + 1, 0, L - 1)
    rw = (pos - lo.astype(pos.dtype)).astype(f32)
    wpad = jnp.zeros((B, 64), f32).at[:, : D * M].set(
        ts2img_weights.reshape(B, D * M))

    run = pl.kernel(
        _body,
        out_type=jax.ShapeDtypeStruct((B, D, 3, S, S), f32),
        mesh=plsc.VectorSubcoreMesh(
            core_axis_name="c", subcore_axis_name="s",
            num_cores=NC, num_subcores=NS),
        compiler_params=pltpu.CompilerParams(needs_layout_passes=False),
        scratch_types=[
            pltpu.VMEM((S,), i32),
            pltpu.VMEM((S,), i32),
            pltpu.VMEM((S,), f32),
            pltpu.VMEM((L * D,), f32),
            pltpu.VMEM((64,), f32),
            pltpu.VMEM((S + LANES,), f32),
            pltpu.VMEM((S + LANES,), f32),
            pltpu.VMEM((S + LANES,), f32),
            pltpu.VMEM((2, 3, S, S), f32),
            pltpu.SemaphoreType.DMA,
        ],
    )
    return run(x.reshape(B, L * D), wpad, lo, hi, rw)
